# baseline (speedup 1.0000x reference)
import functools

import jax
import jax.numpy as jnp
from jax import lax
from jax.experimental import pallas as pl
from jax.experimental.pallas import tpu as pltpu
from jax.experimental.pallas import tpu_sc as plsc

_NSPK = 1000
_EMBED = 128
_BATCH = 16384
_OFFSET = 120
_MOD = 240


_NC = 2
_NS = 16
_NW = _NC * _NS
_BPW = _BATCH // _NW
_CK = 128
_NCHUNK = _BPW // _CK
_NBUF = 3

_mesh = plsc.VectorSubcoreMesh(core_axis_name="c", subcore_axis_name="s")


@functools.partial(
    pl.kernel,
    mesh=_mesh,
    out_type=jax.ShapeDtypeStruct((_BATCH, _EMBED), jnp.float32),
    scratch_types=[
        pltpu.VMEM((_BPW,), jnp.int32),
        pltpu.VMEM((_NBUF, _CK, 2 * _EMBED), jnp.float32),
    ]
    + [pltpu.SemaphoreType.DMA] * (2 * _NBUF),
)
def _sc_gather(idx_hbm, comb_hbm, out, idx_v, bufs, *sems):
    semg = sems[:_NBUF]
    semw = sems[_NBUF:]
    wid = lax.axis_index("s") * _NC + lax.axis_index("c")
    base = wid * _BPW

    pltpu.sync_copy(idx_hbm.at[wid], idx_v)

    def gather(t):
        return pltpu.async_copy(
            comb_hbm.at[idx_v.at[pl.ds(t * _CK, _CK)]],
            bufs.at[t % _NBUF],
            semg[t % _NBUF],
        )

    def write(t):
        s = t % _NBUF
        return pltpu.async_copy(
            bufs.at[s, :, pl.ds(0, _EMBED)],
            out.at[pl.ds(base + t * _CK, _CK)],
            semw[s],
        )

    gh = [None] * _NCHUNK
    wh = [None] * _NCHUNK
    for t in range(_NBUF):
        gh[t] = gather(t)
    for t in range(_NCHUNK):
        nxt = t + _NBUF - 1
        if _NBUF <= nxt < _NCHUNK:
            wh[nxt - _NBUF].wait()
            gh[nxt] = gather(nxt)
        gh[t].wait()
        wh[t] = write(t)
    for t in range(max(0, _NCHUNK - _NBUF), _NCHUNK):
        wh[t].wait()


_K = 256
_BBLK = 1024
_NBLK = _BATCH // _BBLK


def _tc_body(idx_ref, table_ref, out_ref):
    idx = idx_ref[0, 0, :]
    iota = lax.broadcasted_iota(jnp.int32, (_BBLK, _K), 1)
    onehot = (idx[:, None] == iota).astype(jnp.float32)
    out_ref[...] = jnp.dot(
        onehot,
        table_ref[...],
        preferred_element_type=jnp.float32,
        precision=lax.Precision.HIGHEST,
    )


_tc_lookup = pl.pallas_call(
    _tc_body,
    grid=(_NBLK,),
    in_specs=[
        pl.BlockSpec((1, 1, _BBLK), lambda i: (i, 0, 0)),
        pl.BlockSpec((_K, _EMBED), lambda i: (0, 0)),
    ],
    out_specs=pl.BlockSpec((_BBLK, _EMBED), lambda i: (i, 0)),
    out_shape=jax.ShapeDtypeStruct((_BATCH, _EMBED), jnp.float32),
)


def kernel(spkr, table):
    idx2 = spkr.reshape(_NW, _BPW)
    idx3 = spkr.reshape(_NBLK, 1, _BBLK)
    tbl = table[:_MOD]
    roll = jnp.roll(tbl, -_OFFSET, axis=0)
    comb = jnp.concatenate([tbl, roll], axis=1)
    rolled = jnp.pad(roll, ((0, _K - _MOD), (0, 0)))
    ident = _sc_gather(idx2, comb)
    ident_n = _tc_lookup(idx3, rolled)
    return ident, ident_n


# speedup vs baseline: 1.0987x; 1.0987x over previous
import functools

import jax
import jax.numpy as jnp
from jax import lax
from jax.experimental import pallas as pl
from jax.experimental.pallas import tpu as pltpu
from jax.experimental.pallas import tpu_sc as plsc

_NSPK = 1000
_EMBED = 128
_BATCH = 16384
_OFFSET = 120
_MOD = 240


_NC = 2
_NS = 16
_NW = _NC * _NS
_BPW = _BATCH // _NW
_CK = 128
_NCHUNK = _BPW // _CK
_NBUF = 4

_mesh = plsc.VectorSubcoreMesh(core_axis_name="c", subcore_axis_name="s")


@functools.partial(
    pl.kernel,
    mesh=_mesh,
    out_type=jax.ShapeDtypeStruct((_BATCH, _EMBED), jnp.float32),
    scratch_types=[
        pltpu.VMEM((_BPW,), jnp.int32),
        pltpu.VMEM((_NBUF, _CK, _EMBED), jnp.float32),
    ]
    + [pltpu.SemaphoreType.DMA] * (2 * _NBUF),
)
def _sc_gather(idx_hbm, table_hbm, out, idx_v, bufs, *sems):
    semg = sems[:_NBUF]
    semw = sems[_NBUF:]
    wid = lax.axis_index("s") * _NC + lax.axis_index("c")
    base = wid * _BPW

    pltpu.sync_copy(idx_hbm.at[wid], idx_v)

    def gather(t):
        return pltpu.async_copy(
            table_hbm.at[idx_v.at[pl.ds(t * _CK, _CK)]],
            bufs.at[t % _NBUF],
            semg[t % _NBUF],
        )

    def write(t):
        s = t % _NBUF
        return pltpu.async_copy(
            bufs.at[s], out.at[pl.ds(base + t * _CK, _CK)], semw[s]
        )

    gh = [None] * _NCHUNK
    wh = [None] * _NCHUNK
    for t in range(_NBUF):
        gh[t] = gather(t)
    for t in range(_NCHUNK):
        nxt = t + _NBUF - 1
        if _NBUF <= nxt < _NCHUNK:
            wh[nxt - _NBUF].wait()
            gh[nxt] = gather(nxt)
        gh[t].wait()
        wh[t] = write(t)
    for t in range(max(0, _NCHUNK - _NBUF), _NCHUNK):
        wh[t].wait()


_K = 256
_BBLK = 1024
_NBLK = _BATCH // _BBLK


def _tc_body(idx_ref, table_ref, out_ref):
    idx = idx_ref[0, 0, :]
    iota = lax.broadcasted_iota(jnp.int32, (_BBLK, _K), 1)
    onehot = (idx[:, None] == iota).astype(jnp.float32)
    out_ref[...] = jnp.dot(
        onehot,
        table_ref[...],
        preferred_element_type=jnp.float32,
        precision=lax.Precision.HIGHEST,
    )


_tc_lookup = pl.pallas_call(
    _tc_body,
    grid=(_NBLK,),
    in_specs=[
        pl.BlockSpec((1, 1, _BBLK), lambda i: (i, 0, 0)),
        pl.BlockSpec((_K, _EMBED), lambda i: (0, 0)),
    ],
    out_specs=pl.BlockSpec((_BBLK, _EMBED), lambda i: (i, 0)),
    out_shape=jax.ShapeDtypeStruct((_BATCH, _EMBED), jnp.float32),
)


def kernel(spkr, table):
    idx2 = spkr.reshape(_NW, _BPW)
    idx3 = spkr.reshape(_NBLK, 1, _BBLK)
    rolled = jnp.pad(
        jnp.roll(table[:_MOD], -_OFFSET, axis=0), ((0, _K - _MOD), (0, 0))
    )
    ident = _sc_gather(idx2, table)
    ident_n = _tc_lookup(idx3, rolled)
    return ident, ident_n


# speedup vs baseline: 1.1284x; 1.0271x over previous
import functools

import jax
import jax.numpy as jnp
from jax import lax
from jax.experimental import pallas as pl
from jax.experimental.pallas import tpu as pltpu
from jax.experimental.pallas import tpu_sc as plsc

_NSPK = 1000
_EMBED = 128
_BATCH = 16384
_OFFSET = 120
_MOD = 240


_NC = 2
_NS = 16
_NW = _NC * _NS
_BPW = _BATCH // _NW
_CK = 128
_NCHUNK = _BPW // _CK
_NBUF = 4

_mesh = plsc.VectorSubcoreMesh(core_axis_name="c", subcore_axis_name="s")


@functools.partial(
    pl.kernel,
    mesh=_mesh,
    out_type=jax.ShapeDtypeStruct((_BATCH, _EMBED), jnp.float32),
    scratch_types=[
        pltpu.VMEM((_BPW,), jnp.int32),
        pltpu.VMEM((_NBUF, _CK, _EMBED), jnp.float32),
    ]
    + [pltpu.SemaphoreType.DMA] * (2 * _NBUF),
)
def _sc_gather(idx_hbm, table_hbm, out, idx_v, bufs, *sems):
    semg = sems[:_NBUF]
    semw = sems[_NBUF:]
    wid = lax.axis_index("s") * _NC + lax.axis_index("c")
    base = wid * _BPW

    pltpu.sync_copy(idx_hbm.at[wid], idx_v)

    def gather(t):
        return pltpu.async_copy(
            table_hbm.at[idx_v.at[pl.ds(t * _CK, _CK)]],
            bufs.at[t % _NBUF],
            semg[t % _NBUF],
        )

    def write(t):
        s = t % _NBUF
        return pltpu.async_copy(
            bufs.at[s], out.at[pl.ds(base + t * _CK, _CK)], semw[s]
        )

    gh = [None] * _NCHUNK
    wh = [None] * _NCHUNK
    for t in range(_NBUF):
        gh[t] = gather(t)
    for t in range(_NCHUNK):
        nxt = t + _NBUF - 1
        if _NBUF <= nxt < _NCHUNK:
            wh[nxt - _NBUF].wait()
            gh[nxt] = gather(nxt)
        gh[t].wait()
        wh[t] = write(t)
    for t in range(max(0, _NCHUNK - _NBUF), _NCHUNK):
        wh[t].wait()


_K = 256
_BBLK = 2048
_NBLK = _BATCH // _BBLK


def _tc_body(idx_ref, table_ref, out_ref):
    idx = idx_ref[0, 0, :]
    iota = lax.broadcasted_iota(jnp.int32, (_BBLK, _K), 1)
    onehot = (idx[:, None] == iota).astype(jnp.float32)
    out_ref[...] = jnp.dot(
        onehot,
        table_ref[...],
        preferred_element_type=jnp.float32,
        precision=lax.Precision.HIGHEST,
    )


_tc_lookup = pl.pallas_call(
    _tc_body,
    grid=(_NBLK,),
    in_specs=[
        pl.BlockSpec((1, 1, _BBLK), lambda i: (i, 0, 0)),
        pl.BlockSpec((_K, _EMBED), lambda i: (0, 0)),
    ],
    out_specs=pl.BlockSpec((_BBLK, _EMBED), lambda i: (i, 0)),
    out_shape=jax.ShapeDtypeStruct((_BATCH, _EMBED), jnp.float32),
)


def kernel(spkr, table):
    idx2 = spkr.reshape(_NW, _BPW)
    idx3 = spkr.reshape(_NBLK, 1, _BBLK)
    rolled = jnp.pad(
        jnp.roll(table[:_MOD], -_OFFSET, axis=0), ((0, _K - _MOD), (0, 0))
    )
    ident = _sc_gather(idx2, table)
    ident_n = _tc_lookup(idx3, rolled)
    return ident, ident_n
